# Initial kernel scaffold; baseline (speedup 1.0000x reference)
import jax
import jax.numpy as jnp
from jax.experimental import pallas as pl


def kernel(user_index, item_index, edge_index, edge_vals, user_emb, item_emb, W1_0, W2_0, W1_1, W2_1):
    raise NotImplementedError("write your pallas kernel here")


# speedup vs baseline: 2.7340x; 2.7340x over previous
import functools

import jax
import jax.numpy as jnp
from jax import lax
from jax.experimental import pallas as pl
from jax.experimental.pallas import tpu as pltpu
from jax.experimental.pallas import tpu_sc as plsc

NU = 25000
NI = 25000
EE = 800000
DD = 64
BB = 4096

HP = 25088
NP = 2 * HP
JUNK = 25024
CH = 128
NCHUNK = EE // CH
TILES = 16
ROWS_PER_TILE = HP // TILES
ZR = 112
ZCOPIES = 14
QT = BB // 32

_mesh = plsc.VectorSubcoreMesh(core_axis_name="c", subcore_axis_name="s")

_GDN = lax.GatherDimensionNumbers(
    offset_dims=(), collapsed_slice_dims=(0,), start_index_map=(0,))


def _splat(v, j):
    idx = jnp.full((16, 1), j, dtype=jnp.int32)
    return lax.gather(v, idx, _GDN, (1,),
                      mode=lax.GatherScatterMode.PROMISE_IN_BOUNDS)


@functools.partial(
    pl.kernel,
    out_type=jax.ShapeDtypeStruct((NP, DD), jnp.float32),
    mesh=_mesh,
    compiler_params=pltpu.CompilerParams(use_tc_tiling_on_sc=False, needs_layout_passes=False),
    scratch_types=[
        pltpu.VMEM((CH,), jnp.int32),
        pltpu.VMEM((CH,), jnp.int32),
        pltpu.VMEM((CH,), jnp.int32),
        pltpu.VMEM((CH,), jnp.float32),
        pltpu.VMEM((CH, DD), jnp.float32),
        pltpu.VMEM((ZR, DD), jnp.float32),
        pltpu.VMEM_SHARED((HP, DD), jnp.float32),
    ],
)
def _spmm(x_hbm, dst_hbm, src_hbm, val_hbm, rel_hbm,
          dstb, locb, srcb, valb, rows, zbuf, acc):
    c = lax.axis_index("c")
    s = lax.axis_index("s")
    lo = c * NU
    junk16 = lax.iota(jnp.int32, 16) + JUNK

    zero16 = jnp.zeros((16,), jnp.float32)
    for r in range(ZR):
        for k in range(DD // 16):
            zbuf[r, pl.ds(k * 16, 16)] = zero16
    rbase = s * ROWS_PER_TILE
    for z in range(ZCOPIES):
        pltpu.sync_copy(zbuf, acc.at[pl.ds(rbase + z * ZR, ZR)])
    plsc.subcore_barrier()

    nchunks = 390 + jnp.where(s < NCHUNK - 390 * TILES, 1, 0)

    def body(i, carry):
        base = (s + i * TILES) * CH
        pltpu.sync_copy(dst_hbm.at[pl.ds(base, CH)], dstb)
        pltpu.sync_copy(src_hbm.at[pl.ds(base, CH)], srcb)
        pltpu.sync_copy(val_hbm.at[pl.ds(base, CH)], valb)
        for g in range(CH // 16):
            sl = pl.ds(g * 16, 16)
            d = dstb[sl]
            m = (d >= lo) & (d < lo + NU)
            locb[sl] = jnp.where(m, d - lo, junk16)
            sv = srcb[sl]
            srcb[sl] = jnp.where(sv >= NU, sv + (HP - NU), sv)
        pltpu.sync_copy(x_hbm.at[srcb], rows)
        for g in range(CH // 16):
            vg = valb[pl.ds(g * 16, 16)]
            for j in range(16):
                sj = _splat(vg, j)
                r = g * 16 + j
                for k in range(DD // 16):
                    ksl = pl.ds(k * 16, 16)
                    rows[r, ksl] = rows[r, ksl] * sj
        pltpu.sync_copy(rows, acc.at[locb], add=True)
        return carry

    lax.fori_loop(0, nchunks, body, 0)
    plsc.subcore_barrier()
    pltpu.sync_copy(acc.at[pl.ds(rbase, ROWS_PER_TILE)],
                    rel_hbm.at[pl.ds(c * HP + rbase, ROWS_PER_TILE)])


def _dense_body(x_ref, rel_ref, w1_ref, w2_ref, o_ref):
    x = x_ref[...]
    rel = rel_ref[...]
    o = (jnp.dot(x + rel, w1_ref[...], preferred_element_type=jnp.float32)
         + jnp.dot(rel * x, w2_ref[...], preferred_element_type=jnp.float32))
    o_ref[...] = jnp.where(o >= 0, o, 0.2 * o)


RB = 512
_dense = pl.pallas_call(
    _dense_body,
    grid=(pl.cdiv(NP, RB),),
    in_specs=[
        pl.BlockSpec((RB, DD), lambda i: (i, 0)),
        pl.BlockSpec((RB, DD), lambda i: (i, 0)),
        pl.BlockSpec((DD, DD), lambda i: (0, 0)),
        pl.BlockSpec((DD, DD), lambda i: (0, 0)),
    ],
    out_specs=pl.BlockSpec((RB, DD), lambda i: (i, 0)),
    out_shape=jax.ShapeDtypeStruct((NP, DD), jnp.float32),
)


@functools.partial(
    pl.kernel,
    out_type=jax.ShapeDtypeStruct((BB,), jnp.float32),
    mesh=_mesh,
    compiler_params=pltpu.CompilerParams(use_tc_tiling_on_sc=False, needs_layout_passes=False),
    scratch_types=[
        pltpu.VMEM((QT,), jnp.int32),
        pltpu.VMEM((QT,), jnp.int32),
        pltpu.VMEM((QT,), jnp.int32),
        pltpu.VMEM((QT, DD), jnp.float32),
        pltpu.VMEM((QT, DD), jnp.float32),
        pltpu.VMEM((QT, DD), jnp.float32),
        pltpu.VMEM((QT, DD), jnp.float32),
        pltpu.VMEM((QT, DD), jnp.float32),
        pltpu.VMEM((QT, DD), jnp.float32),
        pltpu.VMEM((QT,), jnp.float32),
    ],
)
def _score(uq_hbm, iq_hbm, xp_hbm, o1_hbm, o2_hbm, sc_hbm,
           uq, iq, iqp, eu, ei, p1u, p1i, p2u, p2i, scb):
    c = lax.axis_index("c")
    s = lax.axis_index("s")
    base = (c * TILES + s) * QT
    pltpu.sync_copy(uq_hbm.at[pl.ds(base, QT)], uq)
    pltpu.sync_copy(iq_hbm.at[pl.ds(base, QT)], iq)
    for g in range(QT // 16):
        sl = pl.ds(g * 16, 16)
        iqp[sl] = iq[sl] + HP
    pltpu.sync_copy(xp_hbm.at[uq], eu)
    pltpu.sync_copy(xp_hbm.at[iqp], ei)
    pltpu.sync_copy(o1_hbm.at[uq], p1u)
    pltpu.sync_copy(o1_hbm.at[iqp], p1i)
    pltpu.sync_copy(o2_hbm.at[uq], p2u)
    pltpu.sync_copy(o2_hbm.at[iqp], p2i)
    for g in range(QT // 16):
        qi = lax.iota(jnp.int32, 16) + g * 16

        def dbody(d, a):
            di = jnp.full((16,), d, jnp.int32)
            a = a + plsc.load_gather(eu, [qi, di]) * plsc.load_gather(ei, [qi, di])
            a = a + plsc.load_gather(p1u, [qi, di]) * plsc.load_gather(p1i, [qi, di])
            a = a + plsc.load_gather(p2u, [qi, di]) * plsc.load_gather(p2i, [qi, di])
            return a

        scb[pl.ds(g * 16, 16)] = lax.fori_loop(
            0, DD, dbody, jnp.zeros((16,), jnp.float32))
    pltpu.sync_copy(scb, sc_hbm.at[pl.ds(base, QT)])


def kernel(user_index, item_index, edge_index, edge_vals,
           user_emb, item_emb, W1_0, W2_0, W1_1, W2_1):
    zpad = jnp.zeros((HP - NU, DD), jnp.float32)
    x_p = jnp.concatenate([user_emb, zpad, item_emb, zpad], axis=0)
    dst = edge_index[0]
    src = edge_index[1]
    rel1 = _spmm(x_p, dst, src, edge_vals)
    out1 = _dense(x_p, rel1, W1_0, W2_0)
    rel2 = _spmm(out1, dst, src, edge_vals)
    out2 = _dense(out1, rel2, W1_1, W2_1)
    return _score(user_index, item_index, x_p, out1, out2)


# speedup vs baseline: 4.3845x; 1.6037x over previous
import functools

import jax
import jax.numpy as jnp
from jax import lax
from jax.experimental import pallas as pl
from jax.experimental.pallas import tpu as pltpu
from jax.experimental.pallas import tpu_sc as plsc

NU = 25000
NI = 25000
EE = 800000
DD = 64
BB = 4096

HP = 25088
NP = 2 * HP
JUNK = 25024
CH = 128
TILES = 16
NOUTER = 197
NJ = 2 * NOUTER
EPAD = 6336 * CH
ROWS_PER_TILE = HP // TILES
ZR = 112
ZCOPIES = 14
QT = BB // 32

_mesh = plsc.VectorSubcoreMesh(core_axis_name="c", subcore_axis_name="s")
_params = pltpu.CompilerParams(use_tc_tiling_on_sc=False,
                               needs_layout_passes=False)

_GDN = lax.GatherDimensionNumbers(
    offset_dims=(), collapsed_slice_dims=(0,), start_index_map=(0,))


def _splat(v, j):
    idx = jnp.full((16, 1), j, dtype=jnp.int32)
    return lax.gather(v, idx, _GDN, (1,),
                      mode=lax.GatherScatterMode.PROMISE_IN_BOUNDS)


@functools.partial(
    pl.kernel,
    out_type=jax.ShapeDtypeStruct((NP, DD), jnp.float32),
    mesh=_mesh,
    compiler_params=_params,
    scratch_types=[
        pltpu.VMEM((CH,), jnp.int32),
        pltpu.VMEM((CH,), jnp.int32),
        pltpu.VMEM((CH,), jnp.int32),
        pltpu.VMEM((CH,), jnp.int32),
        pltpu.VMEM((CH,), jnp.float32),
        pltpu.VMEM((CH,), jnp.float32),
        pltpu.VMEM((CH,), jnp.int32),
        pltpu.VMEM((CH,), jnp.int32),
        pltpu.VMEM((CH,), jnp.int32),
        pltpu.VMEM((CH,), jnp.int32),
        pltpu.VMEM((CH,), jnp.float32),
        pltpu.VMEM((CH,), jnp.float32),
        pltpu.VMEM((CH, DD), jnp.float32),
        pltpu.VMEM((CH, DD), jnp.float32),
        pltpu.VMEM((ZR, DD), jnp.float32),
        pltpu.VMEM_SHARED((HP, DD), jnp.float32),
        pltpu.SemaphoreType.DMA,
        pltpu.SemaphoreType.DMA,
        pltpu.SemaphoreType.DMA,
        pltpu.SemaphoreType.DMA,
        pltpu.SemaphoreType.DMA,
        pltpu.SemaphoreType.DMA,
    ],
)
def _spmm(x_hbm, dst_hbm, src_hbm, val_hbm, rel_hbm,
          mdst0, mdst1, msrc0, msrc1, mval0, mval1,
          gsrc0, gsrc1, locb0, locb1, wval0, wval1,
          rows0, rows1, zbuf, acc,
          msem0, msem1, gsem0, gsem1, ssem0, ssem1):
    mdst = (mdst0, mdst1)
    msrc = (msrc0, msrc1)
    mval = (mval0, mval1)
    gsrc = (gsrc0, gsrc1)
    locb = (locb0, locb1)
    wval = (wval0, wval1)
    rows = (rows0, rows1)
    msem = (msem0, msem1)
    gsem = (gsem0, gsem1)
    ssem = (ssem0, ssem1)

    c = lax.axis_index("c")
    s = lax.axis_index("s")
    lo = c * NU
    junk16 = lax.iota(jnp.int32, 16) + JUNK
    zero16i = jnp.zeros((16,), jnp.int32)

    zero16 = jnp.zeros((16,), jnp.float32)
    for r in range(ZR):
        for k in range(DD // 16):
            zbuf[r, pl.ds(k * 16, 16)] = zero16
    rbase = s * ROWS_PER_TILE
    for z in range(ZCOPIES):
        pltpu.sync_copy(zbuf, acc.at[pl.ds(rbase + z * ZR, ZR)])
    plsc.subcore_barrier()

    def issue_meta(j, b):
        base = (s + j * TILES) * CH
        pltpu.async_copy(dst_hbm.at[pl.ds(base, CH)], mdst[b], msem[b])
        pltpu.async_copy(src_hbm.at[pl.ds(base, CH)], msrc[b], msem[b])
        pltpu.async_copy(val_hbm.at[pl.ds(base, CH)], mval[b], msem[b])

    def wait_meta(b):
        base = s * CH
        pltpu.make_async_copy(dst_hbm.at[pl.ds(base, CH)], mdst[b], msem[b]).wait()
        pltpu.make_async_copy(src_hbm.at[pl.ds(base, CH)], msrc[b], msem[b]).wait()
        pltpu.make_async_copy(val_hbm.at[pl.ds(base, CH)], mval[b], msem[b]).wait()

    def compute_stage(b):
        for g in range(CH // 16):
            sl = pl.ds(g * 16, 16)
            d = mdst[b][sl]
            m = (d >= lo) & (d < lo + NU)
            locb[b][sl] = jnp.where(m, d - lo, junk16)
            sv = msrc[b][sl]
            gsrc[b][sl] = jnp.where(sv >= NU, sv + (HP - NU), sv)
            wval[b][sl] = mval[b][sl]

    def multiply_stage(b):
        for g in range(CH // 16):
            vg = wval[b][pl.ds(g * 16, 16)]
            for j16 in range(16):
                sj = _splat(vg, j16)
                r = g * 16 + j16
                for k in range(DD // 16):
                    ksl = pl.ds(k * 16, 16)
                    rows[b][r, ksl] = rows[b][r, ksl] * sj

    for g in range(CH // 16):
        sl = pl.ds(g * 16, 16)
        gsrc[1][sl] = zero16i
        locb[0][sl] = junk16
        locb[1][sl] = junk16
    pltpu.async_copy(rows[0], acc.at[locb[0]], ssem[0], add=True)
    pltpu.async_copy(x_hbm.at[gsrc[1]], rows[1], gsem[1])
    issue_meta(0, 0)
    issue_meta(1, 1)

    def outer(i2, carry):
        for b in (0, 1):
            j = i2 * 2 + b
            wait_meta(b)
            pltpu.make_async_copy(rows[b], acc.at[locb[b]], ssem[b]).wait()
            compute_stage(b)
            pltpu.async_copy(x_hbm.at[gsrc[b]], rows[b], gsem[b])
            o = 1 - b
            pltpu.make_async_copy(x_hbm.at[gsrc[o]], rows[o], gsem[o]).wait()
            multiply_stage(o)
            pltpu.async_copy(rows[o], acc.at[locb[o]], ssem[o], add=True)
            issue_meta(j + 2, b)
        return carry

    lax.fori_loop(0, NOUTER, outer, 0)

    pltpu.make_async_copy(x_hbm.at[gsrc[1]], rows[1], gsem[1]).wait()
    multiply_stage(1)
    pltpu.async_copy(rows[1], acc.at[locb[1]], ssem[1], add=True)
    pltpu.make_async_copy(rows[0], acc.at[locb[0]], ssem[0]).wait()
    pltpu.make_async_copy(rows[1], acc.at[locb[1]], ssem[1]).wait()
    wait_meta(0)
    wait_meta(1)

    plsc.subcore_barrier()
    pltpu.sync_copy(acc.at[pl.ds(rbase, ROWS_PER_TILE)],
                    rel_hbm.at[pl.ds(c * HP + rbase, ROWS_PER_TILE)])


def _dense_body(x_ref, rel_ref, w1_ref, w2_ref, o_ref):
    x = x_ref[...]
    rel = rel_ref[...]
    o = (jnp.dot(x + rel, w1_ref[...], preferred_element_type=jnp.float32)
         + jnp.dot(rel * x, w2_ref[...], preferred_element_type=jnp.float32))
    o_ref[...] = jnp.where(o >= 0, o, 0.2 * o)


RB = 512
_dense = pl.pallas_call(
    _dense_body,
    grid=(pl.cdiv(NP, RB),),
    in_specs=[
        pl.BlockSpec((RB, DD), lambda i: (i, 0)),
        pl.BlockSpec((RB, DD), lambda i: (i, 0)),
        pl.BlockSpec((DD, DD), lambda i: (0, 0)),
        pl.BlockSpec((DD, DD), lambda i: (0, 0)),
    ],
    out_specs=pl.BlockSpec((RB, DD), lambda i: (i, 0)),
    out_shape=jax.ShapeDtypeStruct((NP, DD), jnp.float32),
)


@functools.partial(
    pl.kernel,
    out_type=jax.ShapeDtypeStruct((BB,), jnp.float32),
    mesh=_mesh,
    compiler_params=_params,
    scratch_types=[
        pltpu.VMEM((QT,), jnp.int32),
        pltpu.VMEM((QT,), jnp.int32),
        pltpu.VMEM((QT,), jnp.int32),
        pltpu.VMEM((QT, DD), jnp.float32),
        pltpu.VMEM((QT, DD), jnp.float32),
        pltpu.VMEM((QT, DD), jnp.float32),
        pltpu.VMEM((QT, DD), jnp.float32),
        pltpu.VMEM((QT, DD), jnp.float32),
        pltpu.VMEM((QT, DD), jnp.float32),
        pltpu.VMEM((QT,), jnp.float32),
        pltpu.SemaphoreType.DMA,
    ],
)
def _score(uq_hbm, iq_hbm, xp_hbm, o1_hbm, o2_hbm, sc_hbm,
           uq, iq, iqp, eu, ei, p1u, p1i, p2u, p2i, scb, qsem):
    c = lax.axis_index("c")
    s = lax.axis_index("s")
    base = (c * TILES + s) * QT
    pltpu.async_copy(uq_hbm.at[pl.ds(base, QT)], uq, qsem)
    pltpu.async_copy(iq_hbm.at[pl.ds(base, QT)], iq, qsem)
    pltpu.make_async_copy(uq_hbm.at[pl.ds(base, QT)], uq, qsem).wait()
    pltpu.make_async_copy(iq_hbm.at[pl.ds(base, QT)], iq, qsem).wait()
    for g in range(QT // 16):
        sl = pl.ds(g * 16, 16)
        iqp[sl] = iq[sl] + HP
    pltpu.async_copy(xp_hbm.at[uq], eu, qsem)
    pltpu.async_copy(xp_hbm.at[iqp], ei, qsem)
    pltpu.async_copy(o1_hbm.at[uq], p1u, qsem)
    pltpu.async_copy(o1_hbm.at[iqp], p1i, qsem)
    pltpu.async_copy(o2_hbm.at[uq], p2u, qsem)
    pltpu.async_copy(o2_hbm.at[iqp], p2i, qsem)
    pltpu.make_async_copy(xp_hbm.at[uq], eu, qsem).wait()
    pltpu.make_async_copy(xp_hbm.at[iqp], ei, qsem).wait()
    pltpu.make_async_copy(o1_hbm.at[uq], p1u, qsem).wait()
    pltpu.make_async_copy(o1_hbm.at[iqp], p1i, qsem).wait()
    pltpu.make_async_copy(o2_hbm.at[uq], p2u, qsem).wait()
    pltpu.make_async_copy(o2_hbm.at[iqp], p2i, qsem).wait()
    for g in range(QT // 16):
        qi = lax.iota(jnp.int32, 16) + g * 16

        def dbody(d, a):
            di = jnp.full((16,), d, jnp.int32)
            a = a + plsc.load_gather(eu, [qi, di]) * plsc.load_gather(ei, [qi, di])
            a = a + plsc.load_gather(p1u, [qi, di]) * plsc.load_gather(p1i, [qi, di])
            a = a + plsc.load_gather(p2u, [qi, di]) * plsc.load_gather(p2i, [qi, di])
            return a

        scb[pl.ds(g * 16, 16)] = lax.fori_loop(
            0, DD, dbody, jnp.zeros((16,), jnp.float32))
    pltpu.sync_copy(scb, sc_hbm.at[pl.ds(base, QT)])


def kernel(user_index, item_index, edge_index, edge_vals,
           user_emb, item_emb, W1_0, W2_0, W1_1, W2_1):
    zpad = jnp.zeros((HP - NU, DD), jnp.float32)
    x_p = jnp.concatenate([user_emb, zpad, item_emb, zpad], axis=0)
    npad = EPAD - EE
    dst = jnp.concatenate([edge_index[0], jnp.full((npad,), 2 * NU, jnp.int32)])
    src = jnp.concatenate([edge_index[1], jnp.zeros((npad,), jnp.int32)])
    val = jnp.concatenate([edge_vals, jnp.zeros((npad,), jnp.float32)])
    rel1 = _spmm(x_p, dst, src, val)
    out1 = _dense(x_p, rel1, W1_0, W2_0)
    rel2 = _spmm(out1, dst, src, val)
    out2 = _dense(out1, rel2, W1_1, W2_1)
    return _score(user_index, item_index, x_p, out1, out2)
